# baseline (speedup 1.0000x reference)
import jax
import jax.numpy as jnp
from jax.experimental import pallas as pl


def _broadcast_body(vec_ref, out_ref):
    out_ref[...] = jnp.broadcast_to(vec_ref[...][None], out_ref.shape)


def kernel(sequence, pos_table):
    batch, seq_len = sequence.shape
    hidden = pos_table.shape[1]
    table = pos_table[:seq_len]
    bb = 128
    out = pl.pallas_call(
        _broadcast_body,
        grid=(batch // bb,),
        in_specs=[pl.BlockSpec((seq_len, hidden), lambda i: (0, 0))],
        out_specs=pl.BlockSpec((bb, seq_len, hidden), lambda i: (i, 0, 0)),
        out_shape=jax.ShapeDtypeStruct((batch, seq_len, hidden), jnp.float32),
    )(table)
    return out


# speedup vs baseline: 1.2829x; 1.2829x over previous
import jax
import jax.numpy as jnp
from jax.experimental import pallas as pl


def _broadcast_body(vec_ref, out_ref):
    seq_len = vec_ref.shape[0]
    rep = out_ref.shape[0] // seq_len
    out_ref[...] = jnp.concatenate([vec_ref[...]] * rep, axis=0)


def kernel(sequence, pos_table):
    batch, seq_len = sequence.shape
    hidden = pos_table.shape[1]
    table = pos_table[:seq_len]
    bb = 128
    out = pl.pallas_call(
        _broadcast_body,
        grid=(batch // bb,),
        in_specs=[pl.BlockSpec((seq_len, hidden), lambda i: (0, 0))],
        out_specs=pl.BlockSpec((bb * seq_len, hidden), lambda i: (i, 0)),
        out_shape=jax.ShapeDtypeStruct((batch * seq_len, hidden), jnp.float32),
    )(table)
    return out.reshape(batch, seq_len, hidden)


# speedup vs baseline: 5.3277x; 4.1529x over previous
import jax
import jax.numpy as jnp
from jax.experimental import pallas as pl


def _lane_broadcast_body(vec_ref, out_ref):
    out_ref[...] = jnp.broadcast_to(vec_ref[...], out_ref.shape)


def kernel(sequence, pos_table):
    batch, seq_len = sequence.shape
    hidden = pos_table.shape[1]
    flat = pos_table[:seq_len].reshape(seq_len * hidden, 1)
    rblk = 512
    out = pl.pallas_call(
        _lane_broadcast_body,
        grid=(seq_len * hidden // rblk,),
        in_specs=[pl.BlockSpec((rblk, 1), lambda i: (i, 0))],
        out_specs=pl.BlockSpec((rblk, batch), lambda i: (i, 0)),
        out_shape=jax.ShapeDtypeStruct((seq_len * hidden, batch), jnp.float32),
    )(flat)
    return out.reshape(seq_len, hidden, batch).transpose(2, 0, 1)
